# baseline (speedup 1.0000x reference)
import jax
import jax.numpy as jnp
from jax.experimental import pallas as pl
from jax.experimental.pallas import tpu as pltpu

K1 = 33
N = 4096
D = 256
BR = 256
NB = N // BR
_INF_BITS = 0x7F800000


def _dtm_kernel(xf_ref, xb_ref, loss_ref, sprev_ref, b2s_ref):
    i = pl.program_id(0)
    m = pl.program_id(1)
    xb = xb_ref[0]
    xf = xf_ref[0]

    @pl.when(i == 0)
    def _():
        ones = jnp.ones((1, D), jnp.float32)
        b2s_ref[pl.ds(m, 1), :] = jax.lax.dot_general(
            ones, xf * xf, (((1,), (1,)), ((), ())),
            preferred_element_type=jnp.float32)

    a2 = jnp.sum(xb * xb, axis=1, keepdims=True)
    b2 = b2s_ref[pl.ds(m, 1), :]
    g = jax.lax.dot_general(xb, xf, (((1,), (1,)), ((), ())),
                            preferred_element_type=jnp.float32)
    d2 = jnp.maximum(a2 + b2 - 2.0 * g, 0.0)
    bits = jax.lax.bitcast_convert_type(d2, jnp.int32)

    gbits = bits[:, 0:128]
    for j in range(1, N // 128):
        gbits = jnp.minimum(gbits, bits[:, j * 128:(j + 1) * 128])

    lane = jax.lax.broadcasted_iota(jnp.int32, (BR, 128), 1)
    row = jax.lax.broadcasted_iota(jnp.int32, (BR, 128), 0)
    diag_lane = (i * BR + row) % 128
    gm_nd = jnp.where(lane == diag_lane, _INF_BITS, gbits)
    lo0 = jnp.min(gm_nd, axis=1, keepdims=True)
    hi0 = jnp.max(gbits, axis=1, keepdims=True)

    def probe(carry, mid):
        lo, hi, clo, chi = carry
        mid = jnp.clip(mid, lo, hi - 1)
        cnt = jnp.sum((bits <= mid).astype(jnp.float32), axis=1,
                      keepdims=True)
        ge = cnt >= K1
        return (jnp.where(ge, lo, mid + 1), jnp.where(ge, mid, hi),
                jnp.where(ge, clo, cnt), jnp.where(ge, cnt, chi))

    def cond(carry):
        lo, hi = carry[0], carry[1]
        return jnp.any(lo < hi)

    def body(carry):
        lo, hi, clo, chi = carry
        lo_f = jax.lax.bitcast_convert_type(lo, jnp.float32)
        hi_f = jax.lax.bitcast_convert_type(hi, jnp.float32)
        frac = (K1 - clo) / jnp.maximum(chi - clo, 1.0)
        mid_f = lo_f + frac * (hi_f - lo_f)
        carry = probe(carry, jax.lax.bitcast_convert_type(mid_f, jnp.int32))
        lo, hi = carry[0], carry[1]
        return probe(carry, lo + (hi - lo) // 2)

    clo0 = jnp.zeros((BR, 1), jnp.float32)
    chi0 = jnp.full((BR, 1), 128.0, jnp.float32)
    lo_fin, _, _, _ = jax.lax.while_loop(cond, body, (lo0, hi0, clo0, chi0))
    tbits = lo_fin
    t = jax.lax.bitcast_convert_type(tbits, jnp.float32)

    dist = jnp.sqrt(d2)
    lt = bits < tbits
    cnt_lt = jnp.sum(lt.astype(jnp.float32), axis=1, keepdims=True)
    sum_lt = jnp.sum(jnp.where(lt, dist, 0.0), axis=1, keepdims=True)
    s = sum_lt + (K1 - cnt_lt) * jnp.sqrt(t)

    @pl.when(jnp.logical_and(i == 0, m == 0))
    def _():
        loss_ref[:, :] = jnp.zeros((1, 1), jnp.float32)

    @pl.when(m == 0)
    def _():
        sprev_ref[:, :] = s

    @pl.when(m == 1)
    def _():
        diff = s - sprev_ref[:, :]
        loss_ref[:, :] += jnp.sum(diff * diff).reshape(1, 1)

    @pl.when(jnp.logical_and(i == NB - 1, m == 1))
    def _():
        loss_ref[:, :] = loss_ref[:, :] / N


def kernel(x_1, x_2):
    xs = jnp.stack([x_1, x_2])
    out = pl.pallas_call(
        _dtm_kernel,
        grid=(NB, 2),
        in_specs=[
            pl.BlockSpec((1, N, D), lambda i, m: (m, 0, 0)),
            pl.BlockSpec((1, BR, D), lambda i, m: (m, i, 0)),
        ],
        out_specs=pl.BlockSpec((1, 1), lambda i, m: (0, 0)),
        out_shape=jax.ShapeDtypeStruct((1, 1), jnp.float32),
        scratch_shapes=[pltpu.VMEM((BR, 1), jnp.float32),
                        pltpu.VMEM((2, N), jnp.float32)],
    )(xs, xs)
    return out[0, 0]


# speedup vs baseline: 2.0180x; 2.0180x over previous
import jax
import jax.numpy as jnp
from jax.experimental import pallas as pl
from jax.experimental.pallas import tpu as pltpu

K1 = 33
N = 4096
D = 256
BR = 256
NB = N // BR
_INF_BITS = 0x7F800000


def _dtm_kernel(xf_ref, xb_ref, loss_ref, sprev_ref, b2s_ref):
    i = pl.program_id(0)
    m = pl.program_id(1)
    xb = xb_ref[0]
    xf = xf_ref[0]

    @pl.when(i == 0)
    def _():
        ones = jnp.ones((1, D), jnp.float32)
        b2s_ref[pl.ds(m, 1), :] = jax.lax.dot_general(
            ones, xf * xf, (((1,), (1,)), ((), ())),
            preferred_element_type=jnp.float32)

    a2 = jnp.sum(xb * xb, axis=1, keepdims=True)
    b2 = b2s_ref[pl.ds(m, 1), :]
    g = jax.lax.dot_general(xb, xf, (((1,), (1,)), ((), ())),
                            preferred_element_type=jnp.float32)
    d2 = jnp.maximum(a2 + b2 - 2.0 * g, 0.0)
    bits = jax.lax.bitcast_convert_type(d2, jnp.int32)

    gbits = bits[:, 0:128]
    for j in range(1, N // 128):
        gbits = jnp.minimum(gbits, bits[:, j * 128:(j + 1) * 128])

    lane = jax.lax.broadcasted_iota(jnp.int32, (BR, 128), 1)
    row = jax.lax.broadcasted_iota(jnp.int32, (BR, 128), 0)
    diag_lane = (i * BR + row) % 128
    gm_nd = jnp.where(lane == diag_lane, _INF_BITS, gbits)
    lo0 = jnp.min(gm_nd, axis=1, keepdims=True)
    hi0 = jnp.max(gbits, axis=1, keepdims=True)

    def probe(carry, mid):
        lo, hi, clo, chi = carry
        mid = jnp.clip(mid, lo, hi - 1)
        cnt = jnp.sum((bits <= mid).astype(jnp.float32), axis=1,
                      keepdims=True)
        ge = cnt >= K1
        return (jnp.where(ge, lo, mid + 1), jnp.where(ge, mid, hi),
                jnp.where(ge, clo, cnt), jnp.where(ge, cnt, chi))

    def done(carry):
        lo, hi, clo, chi = carry
        return (lo >= hi) | (chi == float(K1)) | (clo == float(K1 - 1))

    def cond(carry):
        return jnp.any(~done(carry))

    def body(carry):
        dn = done(carry)
        lo, hi, clo, chi = carry
        lo_f = jax.lax.bitcast_convert_type(lo, jnp.float32)
        hi_f = jax.lax.bitcast_convert_type(hi, jnp.float32)
        frac = (K1 - clo) / jnp.maximum(chi - clo, 1.0)
        mid_f = lo_f + frac * (hi_f - lo_f)
        new = probe(carry, jax.lax.bitcast_convert_type(mid_f, jnp.int32))
        new = probe(new, new[0] + (new[1] - new[0]) // 2)
        return tuple(jnp.where(dn, o, n) for o, n in zip(carry, new))

    clo0 = jnp.zeros((BR, 1), jnp.float32)
    chi0 = jnp.full((BR, 1), 128.0, jnp.float32)
    lo_fin, hi_fin, clo_fin, chi_fin = jax.lax.while_loop(
        cond, body, (lo0, hi0, clo0, chi0))
    maxle = jnp.max(jnp.where(bits <= hi_fin, bits, -1), axis=1,
                    keepdims=True)
    minge = jnp.min(jnp.where(bits >= lo_fin, bits, _INF_BITS), axis=1,
                    keepdims=True)
    tbits = jnp.where(lo_fin >= hi_fin, lo_fin,
                      jnp.where(chi_fin == float(K1), maxle, minge))
    t = jax.lax.bitcast_convert_type(tbits, jnp.float32)

    dist = jnp.sqrt(d2)
    lt = bits < tbits
    cnt_lt = jnp.sum(lt.astype(jnp.float32), axis=1, keepdims=True)
    sum_lt = jnp.sum(jnp.where(lt, dist, 0.0), axis=1, keepdims=True)
    s = sum_lt + (K1 - cnt_lt) * jnp.sqrt(t)

    @pl.when(jnp.logical_and(i == 0, m == 0))
    def _():
        loss_ref[:, :] = jnp.zeros((1, 1), jnp.float32)

    @pl.when(m == 0)
    def _():
        sprev_ref[:, :] = s

    @pl.when(m == 1)
    def _():
        diff = s - sprev_ref[:, :]
        loss_ref[:, :] += jnp.sum(diff * diff).reshape(1, 1)

    @pl.when(jnp.logical_and(i == NB - 1, m == 1))
    def _():
        loss_ref[:, :] = loss_ref[:, :] / N


def kernel(x_1, x_2):
    xs = jnp.stack([x_1, x_2])
    out = pl.pallas_call(
        _dtm_kernel,
        grid=(NB, 2),
        in_specs=[
            pl.BlockSpec((1, N, D), lambda i, m: (m, 0, 0)),
            pl.BlockSpec((1, BR, D), lambda i, m: (m, i, 0)),
        ],
        out_specs=pl.BlockSpec((1, 1), lambda i, m: (0, 0)),
        out_shape=jax.ShapeDtypeStruct((1, 1), jnp.float32),
        scratch_shapes=[pltpu.VMEM((BR, 1), jnp.float32),
                        pltpu.VMEM((2, N), jnp.float32)],
    )(xs, xs)
    return out[0, 0]


# speedup vs baseline: 2.0977x; 1.0395x over previous
import jax
import jax.numpy as jnp
from jax.experimental import pallas as pl
from jax.experimental.pallas import tpu as pltpu

K1 = 33
N = 4096
D = 256
BR = 256
NB = N // BR
_INF_BITS = 0x7F800000


def _dtm_kernel(xf_ref, xb_ref, loss_ref, sprev_ref, b2s_ref):
    i = pl.program_id(0)
    m = pl.program_id(1)
    xb = xb_ref[0]
    xf = xf_ref[0]

    @pl.when(i == 0)
    def _():
        ones = jnp.ones((1, D), jnp.float32)
        b2s_ref[pl.ds(m, 1), :] = jax.lax.dot_general(
            ones, xf * xf, (((1,), (1,)), ((), ())),
            preferred_element_type=jnp.float32)

    a2 = jnp.sum(xb * xb, axis=1, keepdims=True)
    b2 = b2s_ref[pl.ds(m, 1), :]
    g = jax.lax.dot_general(xb, xf, (((1,), (1,)), ((), ())),
                            preferred_element_type=jnp.float32)
    d2 = jnp.maximum(a2 + b2 - 2.0 * g, 0.0)
    bits = jax.lax.bitcast_convert_type(d2, jnp.int32)

    slices = [bits[:, j * 128:(j + 1) * 128] for j in range(N // 128)]
    while len(slices) > 1:
        slices = [jnp.minimum(slices[k], slices[k + 1])
                  for k in range(0, len(slices) - 1, 2)] + (
                      [slices[-1]] if len(slices) % 2 else [])
    gbits = slices[0]

    lane = jax.lax.broadcasted_iota(jnp.int32, (BR, 128), 1)
    row = jax.lax.broadcasted_iota(jnp.int32, (BR, 128), 0)
    diag_lane = (i * BR + row) % 128
    gm_nd = jnp.where(lane == diag_lane, _INF_BITS, gbits)
    lo0 = jnp.min(gm_nd, axis=1, keepdims=True)
    hi0 = jnp.max(gbits, axis=1, keepdims=True)

    def probe(carry, mid):
        lo, hi, clo, chi = carry
        mid = jnp.clip(mid, lo, hi - 1)
        cnt = jnp.sum((bits <= mid).astype(jnp.float32), axis=1,
                      keepdims=True)
        ge = cnt >= K1
        return (jnp.where(ge, lo, mid + 1), jnp.where(ge, mid, hi),
                jnp.where(ge, clo, cnt), jnp.where(ge, cnt, chi))

    def done(carry):
        lo, hi, clo, chi = carry
        return (lo >= hi) | (chi == float(K1)) | (clo == float(K1 - 1))

    def cond(carry):
        return jnp.any(~done(carry))

    def body(carry):
        dn = done(carry)
        lo, hi, clo, chi = carry
        lo_f = jax.lax.bitcast_convert_type(lo, jnp.float32)
        hi_f = jax.lax.bitcast_convert_type(hi, jnp.float32)
        frac = (K1 - clo) / jnp.maximum(chi - clo, 1.0)
        mid_f = lo_f + frac * (hi_f - lo_f)
        new = probe(carry, jax.lax.bitcast_convert_type(mid_f, jnp.int32))
        new = probe(new, new[0] + (new[1] - new[0]) // 2)
        return tuple(jnp.where(dn, o, n) for o, n in zip(carry, new))

    clo0 = jnp.zeros((BR, 1), jnp.float32)
    chi0 = jnp.full((BR, 1), 128.0, jnp.float32)
    lo_fin, hi_fin, clo_fin, chi_fin = jax.lax.while_loop(
        cond, body, (lo0, hi0, clo0, chi0))
    direct = chi_fin == float(K1)
    minge = jnp.min(jnp.where(bits >= lo_fin, bits, _INF_BITS), axis=1,
                    keepdims=True)
    tbits = jnp.where(lo_fin >= hi_fin, lo_fin, minge)
    t = jax.lax.bitcast_convert_type(tbits, jnp.float32)

    thr2 = jnp.where(direct, hi_fin, tbits - 1)
    dist = jnp.sqrt(d2)
    le = bits <= thr2
    cnt2 = jnp.sum(le.astype(jnp.float32), axis=1, keepdims=True)
    sum2 = jnp.sum(jnp.where(le, dist, 0.0), axis=1, keepdims=True)
    s = sum2 + jnp.where(direct, 0.0, (K1 - cnt2) * jnp.sqrt(t))

    @pl.when(jnp.logical_and(i == 0, m == 0))
    def _():
        loss_ref[:, :] = jnp.zeros((1, 1), jnp.float32)

    @pl.when(m == 0)
    def _():
        sprev_ref[:, :] = s

    @pl.when(m == 1)
    def _():
        diff = s - sprev_ref[:, :]
        loss_ref[:, :] += jnp.sum(diff * diff).reshape(1, 1)

    @pl.when(jnp.logical_and(i == NB - 1, m == 1))
    def _():
        loss_ref[:, :] = loss_ref[:, :] / N


def kernel(x_1, x_2):
    xs = jnp.stack([x_1, x_2])
    out = pl.pallas_call(
        _dtm_kernel,
        grid=(NB, 2),
        in_specs=[
            pl.BlockSpec((1, N, D), lambda i, m: (m, 0, 0)),
            pl.BlockSpec((1, BR, D), lambda i, m: (m, i, 0)),
        ],
        out_specs=pl.BlockSpec((1, 1), lambda i, m: (0, 0)),
        out_shape=jax.ShapeDtypeStruct((1, 1), jnp.float32),
        scratch_shapes=[pltpu.VMEM((BR, 1), jnp.float32),
                        pltpu.VMEM((2, N), jnp.float32)],
    )(xs, xs)
    return out[0, 0]


# speedup vs baseline: 2.1739x; 1.0363x over previous
import jax
import jax.numpy as jnp
from jax.experimental import pallas as pl
from jax.experimental.pallas import tpu as pltpu

K1 = 33
N = 4096
D = 256
BR = 512
NB = N // BR
_INF_BITS = 0x7F800000


def _dtm_kernel(xf_ref, xb_ref, loss_ref, sprev_ref, b2s_ref):
    i = pl.program_id(0)
    m = pl.program_id(1)
    xb = xb_ref[0]
    xf = xf_ref[0]

    @pl.when(i == 0)
    def _():
        ones = jnp.ones((1, D), jnp.float32)
        b2s_ref[pl.ds(m, 1), :] = jax.lax.dot_general(
            ones, xf * xf, (((1,), (1,)), ((), ())),
            preferred_element_type=jnp.float32)

    a2 = jnp.sum(xb * xb, axis=1, keepdims=True)
    b2 = b2s_ref[pl.ds(m, 1), :]
    g = jax.lax.dot_general(xb, xf, (((1,), (1,)), ((), ())),
                            preferred_element_type=jnp.float32)
    d2 = jnp.maximum(a2 + b2 - 2.0 * g, 0.0)
    bits = jax.lax.bitcast_convert_type(d2, jnp.int32)

    slices = [bits[:, j * 128:(j + 1) * 128] for j in range(N // 128)]
    while len(slices) > 1:
        slices = [jnp.minimum(slices[k], slices[k + 1])
                  for k in range(0, len(slices) - 1, 2)] + (
                      [slices[-1]] if len(slices) % 2 else [])
    gbits = slices[0]

    lane = jax.lax.broadcasted_iota(jnp.int32, (BR, 128), 1)
    row = jax.lax.broadcasted_iota(jnp.int32, (BR, 128), 0)
    diag_lane = (i * BR + row) % 128
    gm_nd = jnp.where(lane == diag_lane, _INF_BITS, gbits)
    lo0 = jnp.min(gm_nd, axis=1, keepdims=True)
    hi0 = jnp.max(gbits, axis=1, keepdims=True)

    def probe(carry, mid):
        lo, hi, clo, chi = carry
        mid = jnp.clip(mid, lo, hi - 1)
        cnt = jnp.sum((bits <= mid).astype(jnp.float32), axis=1,
                      keepdims=True)
        ge = cnt >= K1
        return (jnp.where(ge, lo, mid + 1), jnp.where(ge, mid, hi),
                jnp.where(ge, clo, cnt), jnp.where(ge, cnt, chi))

    def done(carry):
        lo, hi, clo, chi = carry
        return (lo >= hi) | (chi == float(K1)) | (clo == float(K1 - 1))

    def cond(carry):
        return jnp.any(~done(carry))

    def body(carry):
        dn = done(carry)
        lo, hi, clo, chi = carry
        lo_f = jax.lax.bitcast_convert_type(lo, jnp.float32)
        hi_f = jax.lax.bitcast_convert_type(hi, jnp.float32)
        frac = (K1 - clo) / jnp.maximum(chi - clo, 1.0)
        mid_f = lo_f + frac * (hi_f - lo_f)
        new = probe(carry, jax.lax.bitcast_convert_type(mid_f, jnp.int32))
        new = probe(new, new[0] + (new[1] - new[0]) // 2)
        return tuple(jnp.where(dn, o, n) for o, n in zip(carry, new))

    clo0 = jnp.zeros((BR, 1), jnp.float32)
    chi0 = jnp.full((BR, 1), 128.0, jnp.float32)
    lo_fin, hi_fin, clo_fin, chi_fin = jax.lax.while_loop(
        cond, body, (lo0, hi0, clo0, chi0))
    direct = chi_fin == float(K1)
    minge = jnp.min(jnp.where(bits >= lo_fin, bits, _INF_BITS), axis=1,
                    keepdims=True)
    tbits = jnp.where(lo_fin >= hi_fin, lo_fin, minge)
    t = jax.lax.bitcast_convert_type(tbits, jnp.float32)

    thr2 = jnp.where(direct, hi_fin, tbits - 1)
    dist = d2 * jax.lax.rsqrt(jnp.maximum(d2, 1e-30))
    le = bits <= thr2
    cnt2 = jnp.sum(le.astype(jnp.float32), axis=1, keepdims=True)
    sum2 = jnp.sum(jnp.where(le, dist, 0.0), axis=1, keepdims=True)
    s = sum2 + jnp.where(direct, 0.0, (K1 - cnt2) * jnp.sqrt(t))

    @pl.when(jnp.logical_and(i == 0, m == 0))
    def _():
        loss_ref[:, :] = jnp.zeros((1, 1), jnp.float32)

    @pl.when(m == 0)
    def _():
        sprev_ref[:, :] = s

    @pl.when(m == 1)
    def _():
        diff = s - sprev_ref[:, :]
        loss_ref[:, :] += jnp.sum(diff * diff).reshape(1, 1)

    @pl.when(jnp.logical_and(i == NB - 1, m == 1))
    def _():
        loss_ref[:, :] = loss_ref[:, :] / N


def kernel(x_1, x_2):
    xs = jnp.stack([x_1, x_2])
    out = pl.pallas_call(
        _dtm_kernel,
        grid=(NB, 2),
        in_specs=[
            pl.BlockSpec((1, N, D), lambda i, m: (m, 0, 0)),
            pl.BlockSpec((1, BR, D), lambda i, m: (m, i, 0)),
        ],
        out_specs=pl.BlockSpec((1, 1), lambda i, m: (0, 0)),
        out_shape=jax.ShapeDtypeStruct((1, 1), jnp.float32),
        scratch_shapes=[pltpu.VMEM((BR, 1), jnp.float32),
                        pltpu.VMEM((2, N), jnp.float32)],
    )(xs, xs)
    return out[0, 0]


# speedup vs baseline: 2.1863x; 1.0057x over previous
import jax
import jax.numpy as jnp
from jax.experimental import pallas as pl
from jax.experimental.pallas import tpu as pltpu

K1 = 33
N = 4096
D = 256
BR = 256
NB = N // BR
_INF_BITS = 0x7F800000


def _dtm_kernel(xf_ref, xb_ref, loss_ref, sprev_ref, b2s_ref):
    i = pl.program_id(0)
    m = pl.program_id(1)
    xb = xb_ref[0]
    xf = xf_ref[0]

    @pl.when(i == 0)
    def _():
        ones = jnp.ones((1, D), jnp.float32)
        b2s_ref[pl.ds(m, 1), :] = jax.lax.dot_general(
            ones, xf * xf, (((1,), (1,)), ((), ())),
            preferred_element_type=jnp.float32)

    a2 = jnp.sum(xb * xb, axis=1, keepdims=True)
    b2 = b2s_ref[pl.ds(m, 1), :]
    g = jax.lax.dot_general(xb, xf, (((1,), (1,)), ((), ())),
                            preferred_element_type=jnp.float32)
    d2 = jnp.maximum(a2 + b2 - 2.0 * g, 0.0)
    bits = jax.lax.bitcast_convert_type(d2, jnp.int32)

    slices = [bits[:, j * 128:(j + 1) * 128] for j in range(N // 128)]
    while len(slices) > 1:
        slices = [jnp.minimum(slices[k], slices[k + 1])
                  for k in range(0, len(slices) - 1, 2)] + (
                      [slices[-1]] if len(slices) % 2 else [])
    gbits = slices[0]

    lane = jax.lax.broadcasted_iota(jnp.int32, (BR, 128), 1)
    row = jax.lax.broadcasted_iota(jnp.int32, (BR, 128), 0)
    diag_lane = (i * BR + row) % 128
    gm_nd = jnp.where(lane == diag_lane, _INF_BITS, gbits)
    lo0 = jnp.min(gm_nd, axis=1, keepdims=True)
    hi0 = jnp.max(gbits, axis=1, keepdims=True)

    def probe(carry, mid):
        lo, hi, clo, chi = carry
        mid = jnp.clip(mid, lo, hi - 1)
        cnt = jnp.sum((bits <= mid).astype(jnp.float32), axis=1,
                      keepdims=True)
        ge = cnt >= K1
        return (jnp.where(ge, lo, mid + 1), jnp.where(ge, mid, hi),
                jnp.where(ge, clo, cnt), jnp.where(ge, cnt, chi))

    def done(carry):
        lo, hi, clo, chi = carry
        return (lo >= hi) | (chi == float(K1)) | (clo == float(K1 - 1))

    def cond(carry):
        return jnp.any(~done(carry))

    def body(carry):
        dn = done(carry)
        lo, hi, clo, chi = carry
        lo_f = jax.lax.bitcast_convert_type(lo, jnp.float32)
        hi_f = jax.lax.bitcast_convert_type(hi, jnp.float32)
        frac = (K1 - clo) / jnp.maximum(chi - clo, 1.0)
        mid_f = lo_f + frac * (hi_f - lo_f)
        new = probe(carry, jax.lax.bitcast_convert_type(mid_f, jnp.int32))
        new = probe(new, new[0] + (new[1] - new[0]) // 2)
        return tuple(jnp.where(dn, o, n) for o, n in zip(carry, new))

    clo0 = jnp.zeros((BR, 1), jnp.float32)
    chi0 = jnp.full((BR, 1), 128.0, jnp.float32)
    lo_fin, hi_fin, clo_fin, chi_fin = jax.lax.while_loop(
        cond, body, (lo0, hi0, clo0, chi0))
    direct = chi_fin == float(K1)
    minge = jnp.min(jnp.where(bits >= lo_fin, bits, _INF_BITS), axis=1,
                    keepdims=True)
    tbits = jnp.where(lo_fin >= hi_fin, lo_fin, minge)
    t = jax.lax.bitcast_convert_type(tbits, jnp.float32)

    thr2 = jnp.where(direct, hi_fin, tbits - 1)
    dist = d2 * jax.lax.rsqrt(jnp.maximum(d2, 1e-30))
    le = bits <= thr2
    cnt2 = jnp.sum(le.astype(jnp.float32), axis=1, keepdims=True)
    sum2 = jnp.sum(jnp.where(le, dist, 0.0), axis=1, keepdims=True)
    s = sum2 + jnp.where(direct, 0.0, (K1 - cnt2) * jnp.sqrt(t))

    @pl.when(jnp.logical_and(i == 0, m == 0))
    def _():
        loss_ref[:, :] = jnp.zeros((1, 1), jnp.float32)

    @pl.when(m == 0)
    def _():
        sprev_ref[:, :] = s

    @pl.when(m == 1)
    def _():
        diff = s - sprev_ref[:, :]
        loss_ref[:, :] += jnp.sum(diff * diff).reshape(1, 1)

    @pl.when(jnp.logical_and(i == NB - 1, m == 1))
    def _():
        loss_ref[:, :] = loss_ref[:, :] / N


def kernel(x_1, x_2):
    xs = jnp.stack([x_1, x_2])
    out = pl.pallas_call(
        _dtm_kernel,
        grid=(NB, 2),
        in_specs=[
            pl.BlockSpec((1, N, D), lambda i, m: (m, 0, 0)),
            pl.BlockSpec((1, BR, D), lambda i, m: (m, i, 0)),
        ],
        out_specs=pl.BlockSpec((1, 1), lambda i, m: (0, 0)),
        out_shape=jax.ShapeDtypeStruct((1, 1), jnp.float32),
        scratch_shapes=[pltpu.VMEM((BR, 1), jnp.float32),
                        pltpu.VMEM((2, N), jnp.float32)],
    )(xs, xs)
    return out[0, 0]


# speedup vs baseline: 2.2311x; 1.0205x over previous
import jax
import jax.numpy as jnp
from jax.experimental import pallas as pl
from jax.experimental.pallas import tpu as pltpu

K1 = 33
N = 4096
D = 256
BR = 256
NB = N // BR
_INF_BITS = 0x7F800000


def _dtm_kernel(xf_ref, xb_ref, loss_ref, sprev_ref, b2s_ref):
    i = pl.program_id(0)
    m = pl.program_id(1)
    xb = xb_ref[0]
    xf = xf_ref[0]

    @pl.when(i == 0)
    def _():
        ones = jnp.ones((1, D), jnp.float32)
        b2s_ref[pl.ds(m, 1), :] = jax.lax.dot_general(
            ones, xf * xf, (((1,), (1,)), ((), ())),
            preferred_element_type=jnp.float32)

    a2 = jnp.sum(xb * xb, axis=1, keepdims=True)
    b2 = b2s_ref[pl.ds(m, 1), :]
    g = jax.lax.dot_general(xb, xf, (((1,), (1,)), ((), ())),
                            preferred_element_type=jnp.float32)
    d2 = jnp.maximum(a2 + b2 - 2.0 * g, 0.0)
    bits = jax.lax.bitcast_convert_type(d2, jnp.int32)

    slices = [bits[:, j * 128:(j + 1) * 128] for j in range(N // 128)]
    while len(slices) > 1:
        slices = [jnp.minimum(slices[k], slices[k + 1])
                  for k in range(0, len(slices) - 1, 2)] + (
                      [slices[-1]] if len(slices) % 2 else [])
    gbits = slices[0]

    lane = jax.lax.broadcasted_iota(jnp.int32, (BR, 128), 1)
    row = jax.lax.broadcasted_iota(jnp.int32, (BR, 128), 0)
    diag_lane = (i * BR + row) % 128
    gm_nd = jnp.where(lane == diag_lane, _INF_BITS, gbits)
    lo0 = jnp.min(gm_nd, axis=1, keepdims=True)
    hi0 = jnp.max(gbits, axis=1, keepdims=True)

    def probe(carry, mid):
        lo, hi, clo, chi = carry
        mid = jnp.clip(mid, lo, hi - 1)
        cnt = jnp.sum((bits <= mid).astype(jnp.float32), axis=1,
                      keepdims=True)
        ge = cnt >= K1
        return (jnp.where(ge, lo, mid + 1), jnp.where(ge, mid, hi),
                jnp.where(ge, clo, cnt), jnp.where(ge, cnt, chi))

    def done(carry):
        lo, hi, clo, chi = carry
        return (lo >= hi) | (chi == float(K1)) | (clo == float(K1 - 1))

    def cond(carry):
        return jnp.any(~done(carry))

    def body(carry):
        dn = done(carry)
        lo, hi, clo, chi = carry
        lo_f = jax.lax.bitcast_convert_type(lo, jnp.float32)
        hi_f = jax.lax.bitcast_convert_type(hi, jnp.float32)
        frac = (K1 - clo) / jnp.maximum(chi - clo, 1.0)
        mid_f = lo_f + frac * (hi_f - lo_f)
        new = probe(carry, jax.lax.bitcast_convert_type(mid_f, jnp.int32))
        lo2_f = jax.lax.bitcast_convert_type(new[0], jnp.float32)
        hi2_f = jax.lax.bitcast_convert_type(new[1], jnp.float32)
        frac2 = (K1 - new[2]) / jnp.maximum(new[3] - new[2], 1.0)
        mid2_f = lo2_f + frac2 * (hi2_f - lo2_f)
        new = probe(new, jax.lax.bitcast_convert_type(mid2_f, jnp.int32))
        new = probe(new, new[0] + (new[1] - new[0]) // 2)
        return tuple(jnp.where(dn, o, n) for o, n in zip(carry, new))

    clo0 = jnp.zeros((BR, 1), jnp.float32)
    chi0 = jnp.full((BR, 1), 128.0, jnp.float32)
    lo_fin, hi_fin, clo_fin, chi_fin = jax.lax.while_loop(
        cond, body, (lo0, hi0, clo0, chi0))
    direct = chi_fin == float(K1)
    minge = jnp.min(jnp.where(bits >= lo_fin, bits, _INF_BITS), axis=1,
                    keepdims=True)
    tbits = jnp.where(lo_fin >= hi_fin, lo_fin, minge)
    t = jax.lax.bitcast_convert_type(tbits, jnp.float32)

    thr2 = jnp.where(direct, hi_fin, tbits - 1)
    dist = d2 * jax.lax.rsqrt(jnp.maximum(d2, 1e-30))
    le = bits <= thr2
    cnt2 = jnp.sum(le.astype(jnp.float32), axis=1, keepdims=True)
    sum2 = jnp.sum(jnp.where(le, dist, 0.0), axis=1, keepdims=True)
    s = sum2 + jnp.where(direct, 0.0, (K1 - cnt2) * jnp.sqrt(t))

    @pl.when(jnp.logical_and(i == 0, m == 0))
    def _():
        loss_ref[:, :] = jnp.zeros((1, 1), jnp.float32)

    @pl.when(m == 0)
    def _():
        sprev_ref[:, :] = s

    @pl.when(m == 1)
    def _():
        diff = s - sprev_ref[:, :]
        loss_ref[:, :] += jnp.sum(diff * diff).reshape(1, 1)

    @pl.when(jnp.logical_and(i == NB - 1, m == 1))
    def _():
        loss_ref[:, :] = loss_ref[:, :] / N


def kernel(x_1, x_2):
    xs = jnp.stack([x_1, x_2])
    out = pl.pallas_call(
        _dtm_kernel,
        grid=(NB, 2),
        in_specs=[
            pl.BlockSpec((1, N, D), lambda i, m: (m, 0, 0)),
            pl.BlockSpec((1, BR, D), lambda i, m: (m, i, 0)),
        ],
        out_specs=pl.BlockSpec((1, 1), lambda i, m: (0, 0)),
        out_shape=jax.ShapeDtypeStruct((1, 1), jnp.float32),
        scratch_shapes=[pltpu.VMEM((BR, 1), jnp.float32),
                        pltpu.VMEM((2, N), jnp.float32)],
    )(xs, xs)
    return out[0, 0]
